# baseline (device time: 88984 ns/iter reference)
import jax
import jax.numpy as jnp
from jax import lax
from jax.experimental import pallas as pl
from jax.experimental.pallas import tpu as pltpu

N_DEV = 32
E_PER = 2
R_HOPS = N_DEV // 2
L_HOPS = N_DEV - 1 - R_HOPS
STRIDE = 4

PERM = [1, 2, 5, 6, 14, 13, 10, 9, 17, 18, 21, 22, 30, 29, 26, 25,
        24, 27, 28, 31, 23, 20, 19, 16, 8, 11, 12, 15, 7, 4, 3, 0]
INV = [0] * N_DEV
for _k, _m in enumerate(PERM):
    INV[_m] = _k


def kernel(x, router_W, route_idx, expert_W):
    n_tok, d = x.shape
    n_exp = router_W.shape[1]
    h = expert_W.shape[2]

    def body(x_ref, rw_ref, idx_ref, ew_ref, out_ref,
             own_buf, bufR, bufL,
             sendR_sems, recvR_sems, sendL_sems, recvL_sems):
        my_pos = lax.axis_index("i")

        iota_row = lax.broadcasted_iota(jnp.int32, (1, N_DEV), 1)
        perm_row = jnp.zeros((1, N_DEV), jnp.int32)
        inv_row = jnp.zeros((1, N_DEV), jnp.int32)
        for k in range(N_DEV):
            perm_row = jnp.where(iota_row == k, PERM[k], perm_row)
            inv_row = jnp.where(iota_row == k, INV[k], inv_row)

        def lookup(row, idx):
            return jnp.sum(jnp.where(iota_row == idx, row, 0))

        r = lookup(inv_row, my_pos)

        def ring_at(offset):
            return lookup(perm_row, lax.rem(r + offset + N_DEV, N_DEV))

        barrier_sem = pltpu.get_barrier_semaphore()
        for off in range(1, STRIDE + 1):
            for nbr in (ring_at(off), ring_at(-off)):
                pl.semaphore_signal(
                    barrier_sem, inc=1,
                    device_id=(nbr,), device_id_type=pl.DeviceIdType.MESH,
                )
        pl.semaphore_wait(barrier_sem, 2 * STRIDE)

        def send(src_ref, dst_ref, send_sem, recv_sem, dev):
            rdma = pltpu.make_async_remote_copy(
                src_ref=src_ref, dst_ref=dst_ref,
                send_sem=send_sem, recv_sem=recv_sem,
                device_id=(dev,), device_id_type=pl.DeviceIdType.MESH,
            )
            rdma.start()
            return rdma

        def recv_wait(dst_ref, recv_sem):
            rdma = pltpu.make_async_remote_copy(
                src_ref=dst_ref, dst_ref=dst_ref,
                send_sem=recv_sem, recv_sem=recv_sem,
                device_id=(my_pos,), device_id_type=pl.DeviceIdType.MESH,
            )
            rdma.wait_recv()

        own_buf[...] = ew_ref[...].astype(jnp.bfloat16)
        sends = []
        for j in range(1, STRIDE + 1):
            sends.append(send(own_buf, bufR.at[j - 1], sendR_sems.at[j - 1],
                              recvR_sems.at[j - 1], ring_at(j)))
            if j <= L_HOPS:
                sends.append(send(own_buf, bufL.at[j - 1], sendL_sems.at[j - 1],
                                  recvL_sems.at[j - 1], ring_at(-j)))

        xv = x_ref[:, :]
        xv_bf = xv.astype(jnp.bfloat16)
        scores = jnp.dot(xv, rw_ref[:, :], preferred_element_type=jnp.float32)
        s_max = jnp.max(scores, axis=-1, keepdims=True)
        p = jnp.exp(scores - s_max)
        probs = p / jnp.sum(p, axis=-1, keepdims=True)

        e0 = idx_ref[:, 0:1]
        e1 = idx_ref[:, 1:2]
        iota = lax.broadcasted_iota(jnp.int32, (n_tok, n_exp), 1)
        g0 = jnp.sum(jnp.where(iota == e0, probs, 0.0), axis=1, keepdims=True)
        g1 = jnp.sum(jnp.where(iota == e1, probs, 0.0), axis=1, keepdims=True)
        gs = g0 + g1
        w0 = g0 / gs
        w1 = g1 / gs

        def contrib(pair_ref, origin):
            ge0 = origin * E_PER
            ge1 = ge0 + 1
            m0 = (w0 * (e0 == ge0).astype(jnp.float32)
                  + w1 * (e1 == ge0).astype(jnp.float32)).astype(jnp.bfloat16)
            m1 = (w0 * (e0 == ge1).astype(jnp.float32)
                  + w1 * (e1 == ge1).astype(jnp.float32)).astype(jnp.bfloat16)
            xs = jnp.concatenate([xv_bf * m0, xv_bf * m1], axis=1)
            w2d = pair_ref[...].reshape(E_PER * d, h)
            return jnp.dot(xs, w2d, preferred_element_type=jnp.float32)

        out_ref[:, :] = contrib(own_buf, my_pos)

        for hp in range(max(R_HOPS, L_HOPS)):
            dist = hp + 1
            if hp < R_HOPS:
                recv_wait(bufR.at[hp], recvR_sems.at[hp])
                if dist + STRIDE <= R_HOPS:
                    nxt = dist + STRIDE - 1
                    sends.append(send(bufR.at[hp], bufR.at[nxt],
                                      sendR_sems.at[nxt],
                                      recvR_sems.at[nxt], ring_at(STRIDE)))
            if hp < L_HOPS:
                recv_wait(bufL.at[hp], recvL_sems.at[hp])
                if dist + STRIDE <= L_HOPS:
                    nxt = dist + STRIDE - 1
                    sends.append(send(bufL.at[hp], bufL.at[nxt],
                                      sendL_sems.at[nxt],
                                      recvL_sems.at[nxt], ring_at(-STRIDE)))
            if hp < R_HOPS:
                origin = lookup(perm_row, lax.rem(r - dist + N_DEV, N_DEV))
                out_ref[:, :] = out_ref[:, :] + contrib(bufR.at[hp], origin)
            if hp < L_HOPS:
                origin = lookup(perm_row, lax.rem(r + dist, N_DEV))
                out_ref[:, :] = out_ref[:, :] + contrib(bufL.at[hp], origin)

        for s in sends:
            s.wait_send()

    return pl.pallas_call(
        body,
        out_shape=jax.ShapeDtypeStruct((n_tok, h), jnp.float32),
        in_specs=[pl.BlockSpec(memory_space=pltpu.VMEM)] * 4,
        out_specs=pl.BlockSpec(memory_space=pltpu.VMEM),
        scratch_shapes=[
            pltpu.VMEM((E_PER, d, h), jnp.bfloat16),
            pltpu.VMEM((R_HOPS, E_PER, d, h), jnp.bfloat16),
            pltpu.VMEM((L_HOPS, E_PER, d, h), jnp.bfloat16),
            pltpu.SemaphoreType.DMA((R_HOPS,)),
            pltpu.SemaphoreType.DMA((R_HOPS,)),
            pltpu.SemaphoreType.DMA((L_HOPS,)),
            pltpu.SemaphoreType.DMA((L_HOPS,)),
        ],
        compiler_params=pltpu.CompilerParams(collective_id=0),
    )(x, router_W, route_idx, expert_W)


# device time: 54462 ns/iter; 1.6339x vs baseline; 1.6339x over previous
import jax
import jax.numpy as jnp
from jax import lax
from jax.experimental import pallas as pl
from jax.experimental.pallas import tpu as pltpu

N_DEV = 32
E_PER = 2
R_HOPS = N_DEV // 2
L_HOPS = N_DEV - 1 - R_HOPS
STRIDE = 2

PERM = [1, 2, 5, 6, 14, 13, 10, 9, 17, 18, 21, 22, 30, 29, 26, 25,
        24, 27, 28, 31, 23, 20, 19, 16, 8, 11, 12, 15, 7, 4, 3, 0]
INV = [0] * N_DEV
for _k, _m in enumerate(PERM):
    INV[_m] = _k


def kernel(x, router_W, route_idx, expert_W):
    n_tok, d = x.shape
    n_exp = router_W.shape[1]
    h = expert_W.shape[2]

    def body(x_ref, rw_ref, idx_ref, ew_ref, out_ref,
             own_buf, bufR, bufL,
             sendR_sems, recvR_sems, sendL_sems, recvL_sems):
        my_pos = lax.axis_index("i")

        iota_row = lax.broadcasted_iota(jnp.int32, (1, N_DEV), 1)
        perm_row = jnp.zeros((1, N_DEV), jnp.int32)
        inv_row = jnp.zeros((1, N_DEV), jnp.int32)
        for k in range(N_DEV):
            perm_row = jnp.where(iota_row == k, PERM[k], perm_row)
            inv_row = jnp.where(iota_row == k, INV[k], inv_row)

        def lookup(row, idx):
            return jnp.sum(jnp.where(iota_row == idx, row, 0))

        r = lookup(inv_row, my_pos)

        def ring_at(offset):
            return lookup(perm_row, lax.rem(r + offset + N_DEV, N_DEV))

        barrier_sem = pltpu.get_barrier_semaphore()
        for off in range(1, STRIDE + 1):
            for nbr in (ring_at(off), ring_at(-off)):
                pl.semaphore_signal(
                    barrier_sem, inc=1,
                    device_id=(nbr,), device_id_type=pl.DeviceIdType.MESH,
                )
        pl.semaphore_wait(barrier_sem, 2 * STRIDE)

        def send(src_ref, dst_ref, send_sem, recv_sem, dev):
            rdma = pltpu.make_async_remote_copy(
                src_ref=src_ref, dst_ref=dst_ref,
                send_sem=send_sem, recv_sem=recv_sem,
                device_id=(dev,), device_id_type=pl.DeviceIdType.MESH,
            )
            rdma.start()
            return rdma

        def recv_wait(dst_ref, recv_sem):
            rdma = pltpu.make_async_remote_copy(
                src_ref=dst_ref, dst_ref=dst_ref,
                send_sem=recv_sem, recv_sem=recv_sem,
                device_id=(my_pos,), device_id_type=pl.DeviceIdType.MESH,
            )
            rdma.wait_recv()

        own_buf[...] = ew_ref[...].astype(jnp.bfloat16)
        sends = []
        for j in range(1, STRIDE + 1):
            sends.append(send(own_buf, bufR.at[j - 1], sendR_sems.at[j - 1],
                              recvR_sems.at[j - 1], ring_at(j)))
            if j <= L_HOPS:
                sends.append(send(own_buf, bufL.at[j - 1], sendL_sems.at[j - 1],
                                  recvL_sems.at[j - 1], ring_at(-j)))

        xv = x_ref[:, :]
        xv_bf = xv.astype(jnp.bfloat16)
        scores = jnp.dot(xv, rw_ref[:, :], preferred_element_type=jnp.float32)
        s_max = jnp.max(scores, axis=-1, keepdims=True)
        p = jnp.exp(scores - s_max)
        probs = p / jnp.sum(p, axis=-1, keepdims=True)

        e0 = idx_ref[:, 0:1]
        e1 = idx_ref[:, 1:2]
        iota = lax.broadcasted_iota(jnp.int32, (n_tok, n_exp), 1)
        g0 = jnp.sum(jnp.where(iota == e0, probs, 0.0), axis=1, keepdims=True)
        g1 = jnp.sum(jnp.where(iota == e1, probs, 0.0), axis=1, keepdims=True)
        gs = g0 + g1
        w0 = g0 / gs
        w1 = g1 / gs

        def contrib(pair_ref, origin):
            ge0 = origin * E_PER
            ge1 = ge0 + 1
            m0 = (w0 * (e0 == ge0).astype(jnp.float32)
                  + w1 * (e1 == ge0).astype(jnp.float32)).astype(jnp.bfloat16)
            m1 = (w0 * (e0 == ge1).astype(jnp.float32)
                  + w1 * (e1 == ge1).astype(jnp.float32)).astype(jnp.bfloat16)
            xs = jnp.concatenate([xv_bf * m0, xv_bf * m1], axis=1)
            w2d = pair_ref[...].reshape(E_PER * d, h)
            return jnp.dot(xs, w2d, preferred_element_type=jnp.float32)

        out_ref[:, :] = contrib(own_buf, my_pos)

        for hp in range(max(R_HOPS, L_HOPS)):
            dist = hp + 1
            if hp < R_HOPS:
                recv_wait(bufR.at[hp], recvR_sems.at[hp])
                if dist + STRIDE <= R_HOPS:
                    nxt = dist + STRIDE - 1
                    sends.append(send(bufR.at[hp], bufR.at[nxt],
                                      sendR_sems.at[nxt],
                                      recvR_sems.at[nxt], ring_at(STRIDE)))
            if hp < L_HOPS:
                recv_wait(bufL.at[hp], recvL_sems.at[hp])
                if dist + STRIDE <= L_HOPS:
                    nxt = dist + STRIDE - 1
                    sends.append(send(bufL.at[hp], bufL.at[nxt],
                                      sendL_sems.at[nxt],
                                      recvL_sems.at[nxt], ring_at(-STRIDE)))
            if hp < R_HOPS:
                origin = lookup(perm_row, lax.rem(r - dist + N_DEV, N_DEV))
                out_ref[:, :] = out_ref[:, :] + contrib(bufR.at[hp], origin)
            if hp < L_HOPS:
                origin = lookup(perm_row, lax.rem(r + dist, N_DEV))
                out_ref[:, :] = out_ref[:, :] + contrib(bufL.at[hp], origin)

        for s in sends:
            s.wait_send()

    return pl.pallas_call(
        body,
        out_shape=jax.ShapeDtypeStruct((n_tok, h), jnp.float32),
        in_specs=[pl.BlockSpec(memory_space=pltpu.VMEM)] * 4,
        out_specs=pl.BlockSpec(memory_space=pltpu.VMEM),
        scratch_shapes=[
            pltpu.VMEM((E_PER, d, h), jnp.bfloat16),
            pltpu.VMEM((R_HOPS, E_PER, d, h), jnp.bfloat16),
            pltpu.VMEM((L_HOPS, E_PER, d, h), jnp.bfloat16),
            pltpu.SemaphoreType.DMA((R_HOPS,)),
            pltpu.SemaphoreType.DMA((R_HOPS,)),
            pltpu.SemaphoreType.DMA((L_HOPS,)),
            pltpu.SemaphoreType.DMA((L_HOPS,)),
        ],
        compiler_params=pltpu.CompilerParams(collective_id=0),
    )(x, router_W, route_idx, expert_W)
